# baseline (speedup 1.0000x reference)
import functools

import numpy as np
import jax
import jax.numpy as jnp
from jax import lax
from jax.experimental import pallas as pl
from jax.experimental.pallas import tpu as pltpu
from jax.experimental.pallas import tpu_sc as plsc

_POS_DIM = 3
_NR_LEVELS = 16
_NR_FEAT = 2
_LOG2_HASH = 19
_CAPACITY = 2 ** _LOG2_HASH
_N_POINTS = 262144
_SCALES = np.geomspace(1.0, 1e-4, num=_NR_LEVELS).astype(np.float32)

_NC, _NS = 2, 16
_NW = _NC * _NS
_NPT = _N_POINTS // _NW
_C = 2048
_NCHUNK = _NPT // _C
_NSUB = 4
_SUBC = _C // _NSUB
_NGS = _SUBC // 16
_STRIPE = _CAPACITY // _NS

_P1 = np.int32(np.uint32(2654435761))
_P2 = np.int32(805459861)


def _f(v):
    return jnp.full((16,), v, jnp.float32)


def _i(v):
    return jnp.full((16,), v, jnp.int32)


def _bc(s):
    return jnp.broadcast_to(s, (16,))


def _sc_body(px, py, pz, feats, af_hbm, bf_hbm, out_hbm,
             x_v, y_v, z_v, af_v, bf_v,
             idx_0, idx_1, idx_2, idx_3,
             bary_0, bary_1, bary_2, bary_3,
             rows_0, rows_1, rows_2, rows_3,
             f0_v, f1_v, slab, sem_0, sem_1, sem_2, sem_3):
    idx_b = (idx_0, idx_1, idx_2, idx_3)
    bary_b = (bary_0, bary_1, bary_2, bary_3)
    rows_b = (rows_0, rows_1, rows_2, rows_3)
    sem_b = (sem_0, sem_1, sem_2, sem_3)
    cid = lax.axis_index("c")
    sid = lax.axis_index("s")
    wid = sid * _NC + cid
    pbase = wid * _NPT

    pltpu.sync_copy(af_hbm, af_v)
    pltpu.sync_copy(bf_hbm, bf_v)
    pltpu.sync_copy(px.at[pl.ds(pbase, _NPT)], x_v)
    pltpu.sync_copy(py.at[pl.ds(pbase, _NPT)], y_v)
    pltpu.sync_copy(pz.at[pl.ds(pbase, _NPT)], z_v)

    ZERO, ONE, QUARTER, FOUR = _f(0.0), _f(1.0), _f(0.25), _f(4.0)
    TWO = _f(2.0)
    IZERO, IONE = _i(0), _i(1)
    ISH16 = _i(16)
    IMASKHI = _i(np.int32(np.uint32(0xFFFF0000)))
    I3, I4, IM4 = _i(3), _i(4), _i(-4)
    MASK = _i(_CAPACITY - 1)
    P1V, P2V = _i(_P1), _i(_P2)
    lane = lax.iota(jnp.int32, 16)

    def level_body(l, carry):
        s0 = l * _CAPACITY + sid * _STRIPE
        d0 = sid * _STRIPE
        pltpu.sync_copy(feats.at[pl.ds(s0, _STRIPE)], slab.at[pl.ds(d0, _STRIPE)])
        plsc.subcore_barrier()

        lvec = _bc(l)
        a0 = plsc.load_gather(af_v, [lvec])
        a1 = plsc.load_gather(af_v, [lvec + _i(16)])
        a2 = plsc.load_gather(af_v, [lvec + _i(32)])
        b0 = plsc.load_gather(bf_v, [lvec])
        b1 = plsc.load_gather(bf_v, [lvec + _i(16)])
        b2 = plsc.load_gather(bf_v, [lvec + _i(32)])

        def grp_pass(cbase, sub):
            idx_v, bary_v = idx_b[sub], bary_b[sub]
            sbase = cbase + sub * _SUBC

            @plsc.parallel_loop(0, _NGS, 1, unroll=4)
            def grp_body(i):
                o = i * 16
                x = x_v[pl.ds(sbase + o, 16)]
                y = y_v[pl.ds(sbase + o, 16)]
                z = z_v[pl.ds(sbase + o, 16)]
                cf0 = x * a0 + b0
                cf1 = y * a1 + b1
                cf2 = z * a2 + b2
                e = [cf0 + cf1 + cf2,
                     cf1 + cf2 - cf0,
                     cf2 - (cf1 + cf1),
                     -(cf2 + cf2 + cf2)]
                rem0 = []
                for j in range(4):
                    v = e[j] * QUARTER
                    tf = v.astype(jnp.int32).astype(jnp.float32)
                    fl = tf - jnp.where(tf > v, ONE, ZERO)
                    down = fl * FOUR
                    rem0.append(down + jnp.where(e[j] - down > TWO, FOUR, ZERO))
                sum_i = ((rem0[0] + rem0[1] + rem0[2] + rem0[3])
                         * QUARTER).astype(jnp.int32)
                d0_ = [e[j] - rem0[j] for j in range(4)]
                rank = [sum_i, sum_i, sum_i, sum_i]
                for a in range(4):
                    for b in range(a + 1, 4):
                        less = d0_[a] < d0_[b]
                        rank[a] = rank[a] + jnp.where(less, IONE, IZERO)
                        rank[b] = rank[b] + jnp.where(less, IZERO, IONE)
                rem0i = [rem0[j].astype(jnp.int32) for j in range(4)]
                for j in range(4):
                    adj = jnp.where(rank[j] < 0, I4,
                                    jnp.where(rank[j] > I3, IM4, IZERO))
                    rank[j] = rank[j] + adj
                    rem0i[j] = rem0i[j] + adj
                delta = [(e[j] - rem0i[j].astype(jnp.float32)) * QUARTER
                         for j in range(4)]
                s = []
                for c in range(4):
                    cc = _i(c)
                    acc = jnp.where(rank[0] == cc, delta[0], ZERO)
                    for j in range(1, 4):
                        acc = acc + jnp.where(rank[j] == cc, delta[j], ZERO)
                    s.append(acc)
                bary = [ONE + s[3] - s[0], s[2] - s[3], s[1] - s[2],
                        s[0] - s[1]]
                for rem in range(4):
                    if rem == 0:
                        k0, k1, k2 = rem0i[0], rem0i[1], rem0i[2]
                    else:
                        thr = _i(3 - rem)
                        radd, rsub = _i(rem), _i(rem - 4)
                        k0 = rem0i[0] + jnp.where(rank[0] > thr, rsub, radd)
                        k1 = rem0i[1] + jnp.where(rank[1] > thr, rsub, radd)
                        k2 = rem0i[2] + jnp.where(rank[2] > thr, rsub, radd)
                    h = (k0 ^ (k1 * P1V) ^ (k2 * P2V)) & MASK
                    idx_v[pl.ds(rem * _SUBC + o, 16)] = h
                    bary_v[pl.ds(rem * _SUBC + o, 16)] = bary[rem]

        def acc_pass(sub):
            bary_v, rows_v = bary_b[sub], rows_b[sub]

            @plsc.parallel_loop(0, _NGS, 1, unroll=4)
            def acc_body(i):
                o = i * 16
                out0 = ZERO
                out1 = ZERO
                for rem in range(4):
                    w = bary_v[pl.ds(rem * _SUBC + o, 16)]
                    v = rows_v[pl.ds(rem * _SUBC + o, 16)]
                    f0 = plsc.bitcast(jnp.left_shift(v, ISH16), jnp.float32)
                    f1 = plsc.bitcast(v & IMASKHI, jnp.float32)
                    out0 = out0 + w * f0
                    out1 = out1 + w * f1
                f0_v[pl.ds(sub * _SUBC + o, 16)] = out0
                f1_v[pl.ds(sub * _SUBC + o, 16)] = out1

        def chunk_body(ci, carry2):
            cbase = ci * _C
            handles = []
            for sub in range(_NSUB):
                grp_pass(cbase, sub)
                handles.append(pltpu.async_copy(slab.at[idx_b[sub]],
                                                rows_b[sub], sem_b[sub]))
            for sub in range(_NSUB):
                handles[sub].wait()
                acc_pass(sub)

            obase = pbase + cbase
            pltpu.sync_copy(f0_v, out_hbm.at[pl.ds((2 * l) * _N_POINTS
                                                   + obase, _C)])
            pltpu.sync_copy(f1_v, out_hbm.at[pl.ds((2 * l + 1) * _N_POINTS
                                                   + obase, _C)])
            return carry2

        lax.fori_loop(0, _NCHUNK, chunk_body, 0, unroll=False)
        plsc.subcore_barrier()
        return carry

    lax.fori_loop(0, _NR_LEVELS, level_body, 0, unroll=False)


@jax.jit
def _encode(px, py, pz, feats, af, bf):
    mesh = plsc.VectorSubcoreMesh(core_axis_name="c", subcore_axis_name="s")
    fn = functools.partial(
        pl.kernel, mesh=mesh,
        compiler_params=pltpu.CompilerParams(
            needs_layout_passes=False, use_tc_tiling_on_sc=False),
        out_type=jax.ShapeDtypeStruct((32 * _N_POINTS,), jnp.float32),
        scratch_types=[
            pltpu.VMEM((_NPT,), jnp.float32),
            pltpu.VMEM((_NPT,), jnp.float32),
            pltpu.VMEM((_NPT,), jnp.float32),
            pltpu.VMEM((_POS_DIM * _NR_LEVELS,), jnp.float32),
            pltpu.VMEM((_POS_DIM * _NR_LEVELS,), jnp.float32),
            pltpu.VMEM((4 * _SUBC,), jnp.int32),
            pltpu.VMEM((4 * _SUBC,), jnp.int32),
            pltpu.VMEM((4 * _SUBC,), jnp.int32),
            pltpu.VMEM((4 * _SUBC,), jnp.int32),
            pltpu.VMEM((4 * _SUBC,), jnp.float32),
            pltpu.VMEM((4 * _SUBC,), jnp.float32),
            pltpu.VMEM((4 * _SUBC,), jnp.float32),
            pltpu.VMEM((4 * _SUBC,), jnp.float32),
            pltpu.VMEM((4 * _SUBC,), jnp.int32),
            pltpu.VMEM((4 * _SUBC,), jnp.int32),
            pltpu.VMEM((4 * _SUBC,), jnp.int32),
            pltpu.VMEM((4 * _SUBC,), jnp.int32),
            pltpu.VMEM((_C,), jnp.float32),
            pltpu.VMEM((_C,), jnp.float32),
            pltpu.VMEM_SHARED((_CAPACITY,), jnp.int32),
            pltpu.SemaphoreType.DMA,
            pltpu.SemaphoreType.DMA,
            pltpu.SemaphoreType.DMA,
            pltpu.SemaphoreType.DMA,
        ],
    )(_sc_body)
    return fn(px, py, pz, feats, af, bf)


def kernel(points, features, random_shift):
    sf = (1.0 / np.sqrt((np.arange(_POS_DIM) + 1.0)
                        * (np.arange(_POS_DIM) + 2.0))).astype(np.float32)
    af = jnp.asarray((sf[None, :] / _SCALES[:, None]).T.reshape(-1),
                     dtype=jnp.float32)
    bf = (random_shift * sf[None, :]).astype(jnp.float32).T.reshape(-1)
    px = points[:, 0]
    py = points[:, 1]
    pz = points[:, 2]
    f0b = jax.lax.bitcast_convert_type(
        features[:, :, 0].reshape(-1).astype(jnp.bfloat16), jnp.uint16)
    f1b = jax.lax.bitcast_convert_type(
        features[:, :, 1].reshape(-1).astype(jnp.bfloat16), jnp.uint16)
    feats = jax.lax.bitcast_convert_type(
        f0b.astype(jnp.uint32) | (f1b.astype(jnp.uint32) << 16), jnp.int32)
    out = _encode(px, py, pz, feats, af, bf)
    return out.reshape(32, _N_POINTS).T


# speedup vs baseline: 1.0387x; 1.0387x over previous
import functools

import numpy as np
import jax
import jax.numpy as jnp
from jax import lax
from jax.experimental import pallas as pl
from jax.experimental.pallas import tpu as pltpu
from jax.experimental.pallas import tpu_sc as plsc

_POS_DIM = 3
_NR_LEVELS = 16
_NR_FEAT = 2
_LOG2_HASH = 19
_CAPACITY = 2 ** _LOG2_HASH
_N_POINTS = 262144
_SCALES = np.geomspace(1.0, 1e-4, num=_NR_LEVELS).astype(np.float32)

_NC, _NS = 2, 16
_NW = _NC * _NS
_NPT = _N_POINTS // _NW
_C = 1024
_NCHUNK = _NPT // _C
_NSUB = 4
_SUBC = _C // _NSUB
_NGS = _SUBC // 16
_STRIPE = _CAPACITY // _NS

_P1 = np.int32(np.uint32(2654435761))
_P2 = np.int32(805459861)


def _f(v):
    return jnp.full((16,), v, jnp.float32)


def _i(v):
    return jnp.full((16,), v, jnp.int32)


def _bc(s):
    return jnp.broadcast_to(s, (16,))


def _sc_body(px, py, pz, feats, af_hbm, bf_hbm, out_hbm,
             x_v, y_v, z_v, af_v, bf_v,
             idx_0, idx_1, idx_2, idx_3,
             bary_0, bary_1, bary_2, bary_3,
             rows_0, rows_1, rows_2, rows_3,
             f0_v, f1_v, slab, sem_0, sem_1, sem_2, sem_3):
    idx_b = (idx_0, idx_1, idx_2, idx_3)
    bary_b = (bary_0, bary_1, bary_2, bary_3)
    rows_b = (rows_0, rows_1, rows_2, rows_3)
    sem_b = (sem_0, sem_1, sem_2, sem_3)
    cid = lax.axis_index("c")
    sid = lax.axis_index("s")
    wid = sid * _NC + cid
    pbase = wid * _NPT

    pltpu.sync_copy(af_hbm, af_v)
    pltpu.sync_copy(bf_hbm, bf_v)
    pltpu.sync_copy(px.at[pl.ds(pbase, _NPT)], x_v)
    pltpu.sync_copy(py.at[pl.ds(pbase, _NPT)], y_v)
    pltpu.sync_copy(pz.at[pl.ds(pbase, _NPT)], z_v)

    ZERO, ONE, QUARTER, FOUR = _f(0.0), _f(1.0), _f(0.25), _f(4.0)
    TWO = _f(2.0)
    IZERO, IONE = _i(0), _i(1)
    ISH16 = _i(16)
    IMASKHI = _i(np.int32(np.uint32(0xFFFF0000)))
    I3, I4, IM4 = _i(3), _i(4), _i(-4)
    MASK = _i(_CAPACITY - 1)
    P1V, P2V = _i(_P1), _i(_P2)
    lane = lax.iota(jnp.int32, 16)

    def level_body(l, carry):
        s0 = l * _CAPACITY + sid * _STRIPE
        d0 = sid * _STRIPE
        pltpu.sync_copy(feats.at[pl.ds(s0, _STRIPE)], slab.at[pl.ds(d0, _STRIPE)])
        plsc.subcore_barrier()

        lvec = _bc(l)
        a0 = plsc.load_gather(af_v, [lvec])
        a1 = plsc.load_gather(af_v, [lvec + _i(16)])
        a2 = plsc.load_gather(af_v, [lvec + _i(32)])
        b0 = plsc.load_gather(bf_v, [lvec])
        b1 = plsc.load_gather(bf_v, [lvec + _i(16)])
        b2 = plsc.load_gather(bf_v, [lvec + _i(32)])

        def grp_pass(cbase, sub):
            idx_v, bary_v = idx_b[sub], bary_b[sub]
            sbase = cbase + sub * _SUBC

            @plsc.parallel_loop(0, _NGS, 1, unroll=2)
            def grp_body(i):
                o = i * 16
                x = x_v[pl.ds(sbase + o, 16)]
                y = y_v[pl.ds(sbase + o, 16)]
                z = z_v[pl.ds(sbase + o, 16)]
                cf0 = x * a0 + b0
                cf1 = y * a1 + b1
                cf2 = z * a2 + b2
                e = [cf0 + cf1 + cf2,
                     cf1 + cf2 - cf0,
                     cf2 - (cf1 + cf1),
                     -(cf2 + cf2 + cf2)]
                rem0 = []
                for j in range(4):
                    v = e[j] * QUARTER
                    tf = v.astype(jnp.int32).astype(jnp.float32)
                    fl = tf - jnp.where(tf > v, ONE, ZERO)
                    down = fl * FOUR
                    rem0.append(down + jnp.where(e[j] - down > TWO, FOUR, ZERO))
                sum_i = ((rem0[0] + rem0[1] + rem0[2] + rem0[3])
                         * QUARTER).astype(jnp.int32)
                d0_ = [e[j] - rem0[j] for j in range(4)]
                rank = [sum_i, sum_i, sum_i, sum_i]
                for a in range(4):
                    for b in range(a + 1, 4):
                        less = d0_[a] < d0_[b]
                        rank[a] = rank[a] + jnp.where(less, IONE, IZERO)
                        rank[b] = rank[b] + jnp.where(less, IZERO, IONE)
                rem0i = [rem0[j].astype(jnp.int32) for j in range(4)]
                for j in range(4):
                    adj = jnp.where(rank[j] < 0, I4,
                                    jnp.where(rank[j] > I3, IM4, IZERO))
                    rank[j] = rank[j] + adj
                    rem0i[j] = rem0i[j] + adj
                delta = [(e[j] - rem0i[j].astype(jnp.float32)) * QUARTER
                         for j in range(4)]
                s = []
                for c in range(4):
                    cc = _i(c)
                    acc = jnp.where(rank[0] == cc, delta[0], ZERO)
                    for j in range(1, 4):
                        acc = acc + jnp.where(rank[j] == cc, delta[j], ZERO)
                    s.append(acc)
                bary = [ONE + s[3] - s[0], s[2] - s[3], s[1] - s[2],
                        s[0] - s[1]]
                for rem in range(4):
                    if rem == 0:
                        k0, k1, k2 = rem0i[0], rem0i[1], rem0i[2]
                    else:
                        thr = _i(3 - rem)
                        radd, rsub = _i(rem), _i(rem - 4)
                        k0 = rem0i[0] + jnp.where(rank[0] > thr, rsub, radd)
                        k1 = rem0i[1] + jnp.where(rank[1] > thr, rsub, radd)
                        k2 = rem0i[2] + jnp.where(rank[2] > thr, rsub, radd)
                    h = (k0 ^ (k1 * P1V) ^ (k2 * P2V)) & MASK
                    idx_v[pl.ds(rem * _SUBC + o, 16)] = h
                    bary_v[pl.ds(rem * _SUBC + o, 16)] = bary[rem]

        def acc_pass(sub):
            bary_v, rows_v = bary_b[sub], rows_b[sub]

            @plsc.parallel_loop(0, _NGS, 1, unroll=2)
            def acc_body(i):
                o = i * 16
                out0 = ZERO
                out1 = ZERO
                for rem in range(4):
                    w = bary_v[pl.ds(rem * _SUBC + o, 16)]
                    v = rows_v[pl.ds(rem * _SUBC + o, 16)]
                    f0 = plsc.bitcast(jnp.left_shift(v, ISH16), jnp.float32)
                    f1 = plsc.bitcast(v & IMASKHI, jnp.float32)
                    out0 = out0 + w * f0
                    out1 = out1 + w * f1
                f0_v[pl.ds(sub * _SUBC + o, 16)] = out0
                f1_v[pl.ds(sub * _SUBC + o, 16)] = out1

        def chunk_body(ci, carry2):
            cbase = ci * _C
            handles = []
            for sub in range(_NSUB):
                grp_pass(cbase, sub)
                handles.append(pltpu.async_copy(slab.at[idx_b[sub]],
                                                rows_b[sub], sem_b[sub]))
            for sub in range(_NSUB):
                handles[sub].wait()
                acc_pass(sub)

            obase = pbase + cbase
            pltpu.sync_copy(f0_v, out_hbm.at[pl.ds((2 * l) * _N_POINTS
                                                   + obase, _C)])
            pltpu.sync_copy(f1_v, out_hbm.at[pl.ds((2 * l + 1) * _N_POINTS
                                                   + obase, _C)])
            return carry2

        lax.fori_loop(0, _NCHUNK, chunk_body, 0, unroll=False)
        plsc.subcore_barrier()
        return carry

    lax.fori_loop(0, _NR_LEVELS, level_body, 0, unroll=False)


@jax.jit
def _encode(px, py, pz, feats, af, bf):
    mesh = plsc.VectorSubcoreMesh(core_axis_name="c", subcore_axis_name="s")
    fn = functools.partial(
        pl.kernel, mesh=mesh,
        compiler_params=pltpu.CompilerParams(
            needs_layout_passes=False, use_tc_tiling_on_sc=False),
        out_type=jax.ShapeDtypeStruct((32 * _N_POINTS,), jnp.float32),
        scratch_types=[
            pltpu.VMEM((_NPT,), jnp.float32),
            pltpu.VMEM((_NPT,), jnp.float32),
            pltpu.VMEM((_NPT,), jnp.float32),
            pltpu.VMEM((_POS_DIM * _NR_LEVELS,), jnp.float32),
            pltpu.VMEM((_POS_DIM * _NR_LEVELS,), jnp.float32),
            pltpu.VMEM((4 * _SUBC,), jnp.int32),
            pltpu.VMEM((4 * _SUBC,), jnp.int32),
            pltpu.VMEM((4 * _SUBC,), jnp.int32),
            pltpu.VMEM((4 * _SUBC,), jnp.int32),
            pltpu.VMEM((4 * _SUBC,), jnp.float32),
            pltpu.VMEM((4 * _SUBC,), jnp.float32),
            pltpu.VMEM((4 * _SUBC,), jnp.float32),
            pltpu.VMEM((4 * _SUBC,), jnp.float32),
            pltpu.VMEM((4 * _SUBC,), jnp.int32),
            pltpu.VMEM((4 * _SUBC,), jnp.int32),
            pltpu.VMEM((4 * _SUBC,), jnp.int32),
            pltpu.VMEM((4 * _SUBC,), jnp.int32),
            pltpu.VMEM((_C,), jnp.float32),
            pltpu.VMEM((_C,), jnp.float32),
            pltpu.VMEM_SHARED((_CAPACITY,), jnp.int32),
            pltpu.SemaphoreType.DMA,
            pltpu.SemaphoreType.DMA,
            pltpu.SemaphoreType.DMA,
            pltpu.SemaphoreType.DMA,
        ],
    )(_sc_body)
    return fn(px, py, pz, feats, af, bf)


def kernel(points, features, random_shift):
    sf = (1.0 / np.sqrt((np.arange(_POS_DIM) + 1.0)
                        * (np.arange(_POS_DIM) + 2.0))).astype(np.float32)
    af = jnp.asarray((sf[None, :] / _SCALES[:, None]).T.reshape(-1),
                     dtype=jnp.float32)
    bf = (random_shift * sf[None, :]).astype(jnp.float32).T.reshape(-1)
    px = points[:, 0]
    py = points[:, 1]
    pz = points[:, 2]
    f0b = jax.lax.bitcast_convert_type(
        features[:, :, 0].reshape(-1).astype(jnp.bfloat16), jnp.uint16)
    f1b = jax.lax.bitcast_convert_type(
        features[:, :, 1].reshape(-1).astype(jnp.bfloat16), jnp.uint16)
    feats = jax.lax.bitcast_convert_type(
        f0b.astype(jnp.uint32) | (f1b.astype(jnp.uint32) << 16), jnp.int32)
    out = _encode(px, py, pz, feats, af, bf)
    return out.reshape(32, _N_POINTS).T
